# baseline (speedup 1.0000x reference)
import functools

import jax
import jax.numpy as jnp
from jax import lax
from jax.experimental import pallas as pl
from jax.experimental.pallas import tpu as pltpu
from jax.experimental.pallas import tpu_sc as plsc

EPS = 1e-9


@functools.lru_cache(maxsize=None)
def _make_bag(V, D, B, Lh, n_hist):
  info = plsc.get_sparse_core_info()
  NC, NS, LANES = info.num_cores, info.num_subcores, info.num_lanes
  NW = NC * NS
  assert B % NW == 0 and D % LANES == 0
  b_per_w = B // NW
  n_bags = n_hist * b_per_w
  n_idx = n_bags * Lh
  C0 = min(128, Lh)
  C1 = Lh - C0
  assert Lh % 8 == 0

  mesh = plsc.VectorSubcoreMesh(core_axis_name="c", subcore_axis_name="s")
  NBUF = 4
  assert n_bags % NBUF == 0

  @functools.partial(
      pl.kernel,
      mesh=mesh,
      compiler_params=pltpu.CompilerParams(use_tc_tiling_on_sc=False),
      out_type=jax.ShapeDtypeStruct((n_hist * B, D), jnp.float32),
      scratch_types=(
          [pltpu.VMEM((n_idx,), jnp.int32)] +
          [pltpu.VMEM((Lh, D), jnp.float32) for _ in range(NBUF)] +
          [pltpu.VMEM((n_bags, D), jnp.float32)] +
          [pltpu.SemaphoreType.DMA for _ in range(NBUF)]
      ),
  )
  def bag(table, hist, out, *scr):
    idx_v = scr[0]
    bufs = list(scr[1:1 + NBUF])
    outv = scr[1 + NBUF]
    sems = list(scr[2 + NBUF:2 + 2 * NBUF])
    wid = lax.axis_index("s") * NC + lax.axis_index("c")
    for i in range(n_hist):
      pltpu.sync_copy(
          hist.at[pl.ds((i * B + wid * b_per_w) * Lh, b_per_w * Lh)],
          idx_v.at[pl.ds(i * b_per_w * Lh, b_per_w * Lh)],
      )

    def start(g, b):
      off = g * Lh
      buf = bufs[b]
      cps = [pltpu.make_async_copy(
          table.at[idx_v.at[pl.ds(off, C0)]], buf.at[pl.ds(0, C0)], sems[b])]
      if C1:
        cps.append(pltpu.make_async_copy(
            table.at[idx_v.at[pl.ds(off + C0, C1)]],
            buf.at[pl.ds(C0, C1)], sems[b]))
      return cps

    zeros = jnp.zeros((LANES,), jnp.float32)
    half = D // LANES
    ACC = 4

    def accum(g, b):
      buf = bufs[b]
      accs = [[zeros] * ACC for _ in range(half)]
      for r in range(Lh):
        a = r % ACC
        for h in range(half):
          accs[h][a] = accs[h][a] + buf[r, pl.ds(h * LANES, LANES)]
      for h in range(half):
        t = (accs[h][0] + accs[h][1]) + (accs[h][2] + accs[h][3])
        outv[g, pl.ds(h * LANES, LANES)] = t

    for b in range(NBUF):
      for c in start(b, b):
        c.start()

    def loop_body(j, _):
      for b in range(NBUF):
        k = j * NBUF + b
        for c in start(k, b):
          c.wait()
        accum(k, b)

        @pl.when(k + NBUF < n_bags)
        def _(k=k, b=b):
          for c in start(k + NBUF, b):
            c.start()

      return 0

    lax.fori_loop(0, n_bags // NBUF, loop_body, 0)

    for i in range(n_hist):
      pltpu.sync_copy(
          outv.at[pl.ds(i * b_per_w, b_per_w)],
          out.at[pl.ds(i * B + wid * b_per_w, b_per_w), :],
      )

  return bag


@functools.lru_cache(maxsize=None)
def _make_relayout(V, D):
  info = plsc.get_sparse_core_info()
  NC, NS, LANES = info.num_cores, info.num_subcores, info.num_lanes
  NW = NC * NS
  assert D == 2 * LANES
  CB = 128
  n_cb = V // CB
  per_w = -(-n_cb // NW)
  tail_w = V - n_cb * CB

  mesh = plsc.VectorSubcoreMesh(core_axis_name="c", subcore_axis_name="s")
  NBUF = 4

  @functools.partial(
      pl.kernel,
      mesh=mesh,
      compiler_params=pltpu.CompilerParams(needs_layout_passes=False),
      out_type=jax.ShapeDtypeStruct((V * D,), jnp.float32),
      scratch_types=(
          [pltpu.VMEM((D, CB), jnp.float32) for _ in range(NBUF)] +
          [pltpu.VMEM((CB * D,), jnp.float32) for _ in range(NBUF)] +
          [pltpu.VMEM((max(tail_w, 1), D), jnp.float32)] +
          [pltpu.SemaphoreType.DMA for _ in range(2 * NBUF)]
      ),
  )
  def relayout(tt, tail, out, *scratch):
    bufs = list(scratch[0:NBUF])
    sts = list(scratch[NBUF:2 * NBUF])
    bt = scratch[2 * NBUF]
    gis = list(scratch[2 * NBUF + 1:2 * NBUF + 1 + NBUF])
    gos = list(scratch[2 * NBUF + 1 + NBUF:2 * NBUF + 1 + 2 * NBUF])
    wid = lax.axis_index("s") * NC + lax.axis_index("c")
    row_iota = lax.iota(jnp.int32, LANES)
    row_iota2 = row_iota + LANES

    def cb_of(k):
      return wid * per_w + k

    def copy_in(cb, b):
      return pltpu.make_async_copy(
          tt.at[:, pl.ds(cb * CB, CB)], bufs[b], gis[b])

    ones = jnp.full((LANES,), 1, jnp.int32)
    mask_cb = jnp.full((LANES,), CB - 1, jnp.int32)
    lanes16 = jnp.full((LANES,), LANES, jnp.int32)

    def transpose(b, st):
      buf = bufs[b]
      col = row_iota
      for v in range(CB):
        lo = plsc.load_gather(buf, [row_iota, col])
        hi = plsc.load_gather(buf, [row_iota2, col])
        idx = col * D + row_iota
        plsc.store_scatter(st, [idx], lo)
        plsc.store_scatter(st, [idx + lanes16], hi)
        col = (col + ones) & mask_cb

    def copy_out(cb, b):
      return pltpu.make_async_copy(
          sts[b], out.at[pl.ds(cb * CB * D, CB * D)], gos[b])

    n_mine = jnp.minimum(jnp.maximum(n_cb - wid * per_w, 0), per_w)

    @pl.when(n_mine > 0)
    def _():
      for b in range(NBUF):
        @pl.when(b < n_mine)
        def _(b=b):
          copy_in(cb_of(b), b).start()

      def step(k, b):
        @pl.when(k < n_mine)
        def _():
          copy_in(cb_of(k), b).wait()

          @pl.when(k >= NBUF)
          def _():
            copy_out(cb_of(k - NBUF), b).wait()
          transpose(b, sts[b])
          copy_out(cb_of(k), b).start()

          @pl.when(k + NBUF < n_mine)
          def _():
            copy_in(cb_of(k + NBUF), b).start()

      def body(j, _):
        for b in range(NBUF):
          step(j * NBUF + b, b)
        return 0

      lax.fori_loop(0, (per_w + NBUF - 1) // NBUF, body, 0)

      for back in range(1, NBUF + 1):
        k = n_mine - back

        @pl.when(k >= 0)
        def _(k=k):
          for b in range(NBUF):
            @pl.when(k % NBUF == b)
            def _(k=k, b=b):
              copy_out(cb_of(k), b).wait()

    if tail_w:
      @pl.when(wid == NW - 1)
      def _():
        pltpu.make_async_copy(tail, bt, gis[0]).start()
        pltpu.make_async_copy(tail, bt, gis[0]).wait()

        def body(r, _):
          sts[0][pl.ds(r * D, LANES)] = bt[r, pl.ds(0, LANES)]
          sts[0][pl.ds(r * D + LANES, LANES)] = bt[r, pl.ds(LANES, LANES)]
          return 0

        lax.fori_loop(0, tail_w, body, 0)
        pltpu.make_async_copy(
            sts[0].at[pl.ds(0, tail_w * D)],
            out.at[pl.ds(n_cb * CB * D, tail_w * D)], gos[0]).start()
        pltpu.make_async_copy(
            sts[0].at[pl.ds(0, tail_w * D)],
            out.at[pl.ds(n_cb * CB * D, tail_w * D)], gos[0]).wait()

  return relayout


def _epilogue_body(n_hist, B, pooled_ref, lbl_ref, w_ref, b_ref, out_ref):
  loss_sum = jnp.float32(0.0)
  correct = jnp.float32(0.0)
  ptp = jnp.float32(0.0); pfp = jnp.float32(0.0); pfn = jnp.float32(0.0)
  ntp = jnp.float32(0.0); nfp = jnp.float32(0.0); nfn = jnp.float32(0.0)
  eps = jnp.float32(EPS)
  for i in range(n_hist):
    pe = pooled_ref[i]
    sq = jnp.sum(pe * pe, axis=1, keepdims=True)
    normed = pe * lax.rsqrt(jnp.maximum(sq, 1e-12))
    logits = jnp.dot(normed, w_ref[i],
                     preferred_element_type=jnp.float32) + b_ref[i]
    p = jnp.clip(jax.nn.sigmoid(logits), eps, 1.0 - eps)
    lbl = lbl_ref[i]
    loss = -lbl * jnp.log(p) - (1.0 - lbl) * jnp.log(1.0 - p)
    loss_sum = loss_sum + jnp.sum(jnp.mean(loss, axis=0))

    pred_pos = p > 0.5
    is_pos = lbl == 1.0
    f32 = lambda x: jnp.asarray(x, jnp.float32)
    correct = correct + jnp.sum(f32(pred_pos == is_pos))
    ptp = ptp + jnp.sum(f32(jnp.logical_and(is_pos, pred_pos)))
    pfp = pfp + jnp.sum(f32(jnp.logical_and(~is_pos, pred_pos)))
    pfn = pfn + jnp.sum(f32(jnp.logical_and(is_pos, ~pred_pos)))

    pred_neg = p < 0.5
    is_neg = lbl == 0.0
    ntp = ntp + jnp.sum(f32(jnp.logical_and(is_neg, pred_neg)))
    nfp = nfp + jnp.sum(f32(jnp.logical_and(~is_neg, pred_neg)))
    nfn = nfn + jnp.sum(f32(jnp.logical_and(is_neg, ~pred_neg)))

  accuracy = correct / jnp.float32(B * 6 * n_hist)
  pos_recall = ptp / jnp.maximum(eps, ptp + pfn)
  pos_precision = ptp / jnp.maximum(eps, ptp + pfp)
  pos_f1 = 2 * pos_recall * pos_precision / jnp.maximum(
      eps, pos_recall + pos_precision)
  neg_recall = ntp / jnp.maximum(eps, ntp + nfn)
  neg_precision = ntp / jnp.maximum(eps, ntp + nfp)
  neg_f1 = 2 * neg_recall * neg_precision / jnp.maximum(
      eps, neg_recall + neg_precision)
  out_ref[0] = loss_sum
  out_ref[1] = (pos_f1 + neg_f1) / 2.0
  out_ref[2] = accuracy


def _epilogue_call(pooled, lbls, Ws, bs):
  n_hist, B, _ = pooled.shape
  return pl.pallas_call(
      functools.partial(_epilogue_body, n_hist, B),
      out_shape=jax.ShapeDtypeStruct((3,), jnp.float32),
      out_specs=pl.BlockSpec(memory_space=pltpu.SMEM),
  )(pooled, lbls, Ws, bs)


def kernel(unique_emb, history_0, history_1, history_2,
           label_0, label_1, label_2,
           W_0, W_1, W_2, b_0, b_1, b_2):
  V, D = unique_emb.shape
  B, Lh = history_0.shape
  hist = jnp.stack([history_0, history_1, history_2]).reshape(3 * B * Lh)
  n_full = (V // 128) * 128
  tail = lax.slice(unique_emb, (n_full, 0), (V, D))
  table = _make_relayout(V, D)(unique_emb.T, tail).reshape(V, D)
  pooled = _make_bag(V, D, B, Lh, 3)(table, hist).reshape(3, B, D)
  lbls = jnp.stack([label_0, label_1, label_2])
  Ws = jnp.stack([W_0, W_1, W_2])
  bs = jnp.stack([b_0, b_1, b_2])[:, None, :]
  o = _epilogue_call(pooled, lbls, Ws, bs)
  return (o[0], o[1], o[2])


# speedup vs baseline: 1.4970x; 1.4970x over previous
import functools

import jax
import jax.numpy as jnp
from jax import lax
from jax.experimental import pallas as pl
from jax.experimental.pallas import tpu as pltpu
from jax.experimental.pallas import tpu_sc as plsc

EPS = 1e-9


@functools.lru_cache(maxsize=None)
def _make_bag(V, D, B, Lh, n_hist):
  info = plsc.get_sparse_core_info()
  NC, NS, LANES = info.num_cores, info.num_subcores, info.num_lanes
  NW = NC * NS
  assert B % NW == 0 and D % LANES == 0
  b_per_w = B // NW
  n_bags = n_hist * b_per_w
  n_idx = n_bags * Lh
  C0 = min(128, Lh)
  C1 = Lh - C0
  assert Lh % 8 == 0

  mesh = plsc.VectorSubcoreMesh(core_axis_name="c", subcore_axis_name="s")
  NBUF = 4
  assert n_bags % NBUF == 0

  @functools.partial(
      pl.kernel,
      mesh=mesh,
      compiler_params=pltpu.CompilerParams(use_tc_tiling_on_sc=False),
      out_type=jax.ShapeDtypeStruct((n_hist * B, D), jnp.float32),
      scratch_types=(
          [pltpu.VMEM((n_idx,), jnp.int32)] +
          [pltpu.VMEM((Lh, D), jnp.float32) for _ in range(NBUF)] +
          [pltpu.VMEM((n_bags, D), jnp.float32)] +
          [pltpu.SemaphoreType.DMA for _ in range(NBUF)]
      ),
  )
  def bag(table, hist, out, *scr):
    idx_v = scr[0]
    bufs = list(scr[1:1 + NBUF])
    outv = scr[1 + NBUF]
    sems = list(scr[2 + NBUF:2 + 2 * NBUF])
    wid = lax.axis_index("s") * NC + lax.axis_index("c")
    for i in range(n_hist):
      pltpu.sync_copy(
          hist.at[pl.ds((i * B + wid * b_per_w) * Lh, b_per_w * Lh)],
          idx_v.at[pl.ds(i * b_per_w * Lh, b_per_w * Lh)],
      )

    def start(g, b):
      off = g * Lh
      buf = bufs[b]
      cps = [pltpu.make_async_copy(
          table.at[idx_v.at[pl.ds(off, C0)]], buf.at[pl.ds(0, C0)], sems[b])]
      if C1:
        cps.append(pltpu.make_async_copy(
            table.at[idx_v.at[pl.ds(off + C0, C1)]],
            buf.at[pl.ds(C0, C1)], sems[b]))
      return cps

    half = D // LANES

    def accum(g, b):
      buf = bufs[b]
      nacc = Lh // 4

      def body(j, carry):
        r = j * 4
        out_c = []
        for h in range(half):
          s = h * LANES
          v = ((buf[r, pl.ds(s, LANES)] + buf[r + 1, pl.ds(s, LANES)]) +
               (buf[r + 2, pl.ds(s, LANES)] + buf[r + 3, pl.ds(s, LANES)]))
          out_c.append(carry[h] + v)
        return tuple(out_c)

      acc = lax.fori_loop(
          0, nacc, body,
          tuple(jnp.zeros((LANES,), jnp.float32) for _ in range(half)))
      for h in range(half):
        outv[g, pl.ds(h * LANES, LANES)] = acc[h]

    for b in range(NBUF):
      for c in start(b, b):
        c.start()

    def loop_body(j, _):
      for b in range(NBUF):
        k = j * NBUF + b
        for c in start(k, b):
          c.wait()
        accum(k, b)

        @pl.when(k + NBUF < n_bags)
        def _(k=k, b=b):
          for c in start(k + NBUF, b):
            c.start()

      return 0

    lax.fori_loop(0, n_bags // NBUF, loop_body, 0)

    for i in range(n_hist):
      pltpu.sync_copy(
          outv.at[pl.ds(i * b_per_w, b_per_w)],
          out.at[pl.ds(i * B + wid * b_per_w, b_per_w), :],
      )

  return bag


@functools.lru_cache(maxsize=None)
def _make_relayout(V, D):
  info = plsc.get_sparse_core_info()
  NC, NS, LANES = info.num_cores, info.num_subcores, info.num_lanes
  NW = NC * NS
  assert D == 2 * LANES
  CB = 128
  n_cb = V // CB
  per_w = -(-n_cb // NW)
  tail_w = V - n_cb * CB

  mesh = plsc.VectorSubcoreMesh(core_axis_name="c", subcore_axis_name="s")
  NBUF = 4

  @functools.partial(
      pl.kernel,
      mesh=mesh,
      compiler_params=pltpu.CompilerParams(needs_layout_passes=False),
      out_type=jax.ShapeDtypeStruct((V * D,), jnp.float32),
      scratch_types=(
          [pltpu.VMEM((D, CB), jnp.float32) for _ in range(NBUF)] +
          [pltpu.VMEM((CB * D,), jnp.float32) for _ in range(NBUF)] +
          [pltpu.VMEM((max(tail_w, 1), D), jnp.float32)] +
          [pltpu.SemaphoreType.DMA for _ in range(2 * NBUF)]
      ),
  )
  def relayout(tt, tail, out, *scratch):
    bufs = list(scratch[0:NBUF])
    sts = list(scratch[NBUF:2 * NBUF])
    bt = scratch[2 * NBUF]
    gis = list(scratch[2 * NBUF + 1:2 * NBUF + 1 + NBUF])
    gos = list(scratch[2 * NBUF + 1 + NBUF:2 * NBUF + 1 + 2 * NBUF])
    wid = lax.axis_index("s") * NC + lax.axis_index("c")
    row_iota = lax.iota(jnp.int32, LANES)
    row_iota2 = row_iota + LANES

    def cb_of(k):
      return wid * per_w + k

    def copy_in(cb, b):
      return pltpu.make_async_copy(
          tt.at[:, pl.ds(cb * CB, CB)], bufs[b], gis[b])

    ones = jnp.full((LANES,), 1, jnp.int32)
    mask_cb = jnp.full((LANES,), CB - 1, jnp.int32)
    lanes16 = jnp.full((LANES,), LANES, jnp.int32)

    def transpose(b, st):
      buf = bufs[b]
      col = row_iota
      for v in range(CB):
        lo = plsc.load_gather(buf, [row_iota, col])
        hi = plsc.load_gather(buf, [row_iota2, col])
        idx = col * D + row_iota
        plsc.store_scatter(st, [idx], lo)
        plsc.store_scatter(st, [idx + lanes16], hi)
        col = (col + ones) & mask_cb

    def copy_out(cb, b):
      return pltpu.make_async_copy(
          sts[b], out.at[pl.ds(cb * CB * D, CB * D)], gos[b])

    n_mine = jnp.minimum(jnp.maximum(n_cb - wid * per_w, 0), per_w)

    @pl.when(n_mine > 0)
    def _():
      for b in range(NBUF):
        @pl.when(b < n_mine)
        def _(b=b):
          copy_in(cb_of(b), b).start()

      def step(k, b):
        @pl.when(k < n_mine)
        def _():
          copy_in(cb_of(k), b).wait()

          @pl.when(k >= NBUF)
          def _():
            copy_out(cb_of(k - NBUF), b).wait()
          transpose(b, sts[b])
          copy_out(cb_of(k), b).start()

          @pl.when(k + NBUF < n_mine)
          def _():
            copy_in(cb_of(k + NBUF), b).start()

      def body(j, _):
        for b in range(NBUF):
          step(j * NBUF + b, b)
        return 0

      lax.fori_loop(0, (per_w + NBUF - 1) // NBUF, body, 0)

      for back in range(1, NBUF + 1):
        k = n_mine - back

        @pl.when(k >= 0)
        def _(k=k):
          for b in range(NBUF):
            @pl.when(k % NBUF == b)
            def _(k=k, b=b):
              copy_out(cb_of(k), b).wait()

    if tail_w:
      @pl.when(wid == NW - 1)
      def _():
        pltpu.make_async_copy(tail, bt, gis[0]).start()
        pltpu.make_async_copy(tail, bt, gis[0]).wait()

        def body(r, _):
          sts[0][pl.ds(r * D, LANES)] = bt[r, pl.ds(0, LANES)]
          sts[0][pl.ds(r * D + LANES, LANES)] = bt[r, pl.ds(LANES, LANES)]
          return 0

        lax.fori_loop(0, tail_w, body, 0)
        pltpu.make_async_copy(
            sts[0].at[pl.ds(0, tail_w * D)],
            out.at[pl.ds(n_cb * CB * D, tail_w * D)], gos[0]).start()
        pltpu.make_async_copy(
            sts[0].at[pl.ds(0, tail_w * D)],
            out.at[pl.ds(n_cb * CB * D, tail_w * D)], gos[0]).wait()

  return relayout


def _epilogue_body(n_hist, B, pooled_ref, lbl_ref, w_ref, b_ref, out_ref):
  loss_sum = jnp.float32(0.0)
  correct = jnp.float32(0.0)
  ptp = jnp.float32(0.0); pfp = jnp.float32(0.0); pfn = jnp.float32(0.0)
  ntp = jnp.float32(0.0); nfp = jnp.float32(0.0); nfn = jnp.float32(0.0)
  eps = jnp.float32(EPS)
  for i in range(n_hist):
    pe = pooled_ref[i]
    sq = jnp.sum(pe * pe, axis=1, keepdims=True)
    normed = pe * lax.rsqrt(jnp.maximum(sq, 1e-12))
    logits = jnp.dot(normed, w_ref[i],
                     preferred_element_type=jnp.float32) + b_ref[i]
    p = jnp.clip(jax.nn.sigmoid(logits), eps, 1.0 - eps)
    lbl = lbl_ref[i]
    loss = -lbl * jnp.log(p) - (1.0 - lbl) * jnp.log(1.0 - p)
    loss_sum = loss_sum + jnp.sum(jnp.mean(loss, axis=0))

    pred_pos = p > 0.5
    is_pos = lbl == 1.0
    f32 = lambda x: jnp.asarray(x, jnp.float32)
    correct = correct + jnp.sum(f32(pred_pos == is_pos))
    ptp = ptp + jnp.sum(f32(jnp.logical_and(is_pos, pred_pos)))
    pfp = pfp + jnp.sum(f32(jnp.logical_and(~is_pos, pred_pos)))
    pfn = pfn + jnp.sum(f32(jnp.logical_and(is_pos, ~pred_pos)))

    pred_neg = p < 0.5
    is_neg = lbl == 0.0
    ntp = ntp + jnp.sum(f32(jnp.logical_and(is_neg, pred_neg)))
    nfp = nfp + jnp.sum(f32(jnp.logical_and(~is_neg, pred_neg)))
    nfn = nfn + jnp.sum(f32(jnp.logical_and(is_neg, ~pred_neg)))

  accuracy = correct / jnp.float32(B * 6 * n_hist)
  pos_recall = ptp / jnp.maximum(eps, ptp + pfn)
  pos_precision = ptp / jnp.maximum(eps, ptp + pfp)
  pos_f1 = 2 * pos_recall * pos_precision / jnp.maximum(
      eps, pos_recall + pos_precision)
  neg_recall = ntp / jnp.maximum(eps, ntp + nfn)
  neg_precision = ntp / jnp.maximum(eps, ntp + nfp)
  neg_f1 = 2 * neg_recall * neg_precision / jnp.maximum(
      eps, neg_recall + neg_precision)
  out_ref[0] = loss_sum
  out_ref[1] = (pos_f1 + neg_f1) / 2.0
  out_ref[2] = accuracy


def _epilogue_call(pooled, lbls, Ws, bs):
  n_hist, B, _ = pooled.shape
  return pl.pallas_call(
      functools.partial(_epilogue_body, n_hist, B),
      out_shape=jax.ShapeDtypeStruct((3,), jnp.float32),
      out_specs=pl.BlockSpec(memory_space=pltpu.SMEM),
  )(pooled, lbls, Ws, bs)


def kernel(unique_emb, history_0, history_1, history_2,
           label_0, label_1, label_2,
           W_0, W_1, W_2, b_0, b_1, b_2):
  V, D = unique_emb.shape
  B, Lh = history_0.shape
  hist = jnp.stack([history_0, history_1, history_2]).reshape(3 * B * Lh)
  n_full = (V // 128) * 128
  tail = lax.slice(unique_emb, (n_full, 0), (V, D))
  table = _make_relayout(V, D)(unique_emb.T, tail).reshape(V, D)
  pooled = _make_bag(V, D, B, Lh, 3)(table, hist).reshape(3, B, D)
  lbls = jnp.stack([label_0, label_1, label_2])
  Ws = jnp.stack([W_0, W_1, W_2])
  bs = jnp.stack([b_0, b_1, b_2])[:, None, :]
  o = _epilogue_call(pooled, lbls, Ws, bs)
  return (o[0], o[1], o[2])


# speedup vs baseline: 2.1900x; 1.4629x over previous
import functools

import jax
import jax.numpy as jnp
from jax import lax
from jax.experimental import pallas as pl
from jax.experimental.pallas import tpu as pltpu
from jax.experimental.pallas import tpu_sc as plsc

EPS = 1e-9


@functools.lru_cache(maxsize=None)
def _make_bag(V, D, B, Lh, n_hist):
  info = plsc.get_sparse_core_info()
  NC, NS, LANES = info.num_cores, info.num_subcores, info.num_lanes
  NW = NC * NS
  assert B % NW == 0 and D % LANES == 0
  b_per_w = B // NW
  n_bags = n_hist * b_per_w
  n_idx = n_bags * Lh
  C0 = min(128, Lh)
  C1 = Lh - C0
  assert Lh % 8 == 0

  mesh = plsc.VectorSubcoreMesh(core_axis_name="c", subcore_axis_name="s")
  NBUF = 4
  assert n_bags % NBUF == 0

  @functools.partial(
      pl.kernel,
      mesh=mesh,
      compiler_params=pltpu.CompilerParams(use_tc_tiling_on_sc=False),
      out_type=jax.ShapeDtypeStruct((n_hist * B, D), jnp.float32),
      scratch_types=(
          [pltpu.VMEM((n_idx,), jnp.int32)] +
          [pltpu.VMEM((Lh, D), jnp.float32) for _ in range(NBUF)] +
          [pltpu.VMEM((n_bags, D), jnp.float32)] +
          [pltpu.SemaphoreType.DMA for _ in range(NBUF)]
      ),
  )
  def bag(table, hist, out, *scr):
    idx_v = scr[0]
    bufs = list(scr[1:1 + NBUF])
    outv = scr[1 + NBUF]
    sems = list(scr[2 + NBUF:2 + 2 * NBUF])
    wid = lax.axis_index("s") * NC + lax.axis_index("c")
    for i in range(n_hist):
      pltpu.sync_copy(
          hist.at[pl.ds((i * B + wid * b_per_w) * Lh, b_per_w * Lh)],
          idx_v.at[pl.ds(i * b_per_w * Lh, b_per_w * Lh)],
      )

    def start(g, b):
      off = g * Lh
      buf = bufs[b]
      cps = [pltpu.make_async_copy(
          table.at[idx_v.at[pl.ds(off, C0)]], buf.at[pl.ds(0, C0)], sems[b])]
      if C1:
        cps.append(pltpu.make_async_copy(
            table.at[idx_v.at[pl.ds(off + C0, C1)]],
            buf.at[pl.ds(C0, C1)], sems[b]))
      return cps

    half = D // LANES

    def accum(g, b):
      buf = bufs[b]
      nacc = Lh // 4

      def body(j, carry):
        r = j * 4
        out_c = []
        for h in range(half):
          s = h * LANES
          v = ((buf[r, pl.ds(s, LANES)] + buf[r + 1, pl.ds(s, LANES)]) +
               (buf[r + 2, pl.ds(s, LANES)] + buf[r + 3, pl.ds(s, LANES)]))
          out_c.append(carry[h] + v)
        return tuple(out_c)

      acc = lax.fori_loop(
          0, nacc, body,
          tuple(jnp.zeros((LANES,), jnp.float32) for _ in range(half)))
      for h in range(half):
        outv[g, pl.ds(h * LANES, LANES)] = acc[h]

    for b in range(NBUF):
      for c in start(b, b):
        c.start()

    def loop_body(j, _):
      for b in range(NBUF):
        k = j * NBUF + b
        for c in start(k, b):
          c.wait()
        accum(k, b)

        @pl.when(k + NBUF < n_bags)
        def _(k=k, b=b):
          for c in start(k + NBUF, b):
            c.start()

      return 0

    lax.fori_loop(0, n_bags // NBUF, loop_body, 0)

    for i in range(n_hist):
      pltpu.sync_copy(
          outv.at[pl.ds(i * b_per_w, b_per_w)],
          out.at[pl.ds(i * B + wid * b_per_w, b_per_w), :],
      )

  return bag


@functools.lru_cache(maxsize=None)
def _make_relayout(V, D):
  info = plsc.get_sparse_core_info()
  NC, NS, LANES = info.num_cores, info.num_subcores, info.num_lanes
  NW = NC * NS
  assert D == 2 * LANES
  CB = 128
  n_cb = V // CB
  per_w = -(-n_cb // NW)
  tail_w = V - n_cb * CB

  mesh = plsc.VectorSubcoreMesh(core_axis_name="c", subcore_axis_name="s")
  NBUF = 4

  @functools.partial(
      pl.kernel,
      mesh=mesh,
      compiler_params=pltpu.CompilerParams(needs_layout_passes=False),
      out_type=jax.ShapeDtypeStruct((V * D,), jnp.float32),
      scratch_types=(
          [pltpu.VMEM((D, CB), jnp.float32) for _ in range(NBUF)] +
          [pltpu.VMEM((CB * D,), jnp.float32) for _ in range(NBUF)] +
          [pltpu.VMEM((max(tail_w, 1), D), jnp.float32)] +
          [pltpu.SemaphoreType.DMA for _ in range(2 * NBUF)]
      ),
  )
  def relayout(tt, tail, out, *scratch):
    bufs = list(scratch[0:NBUF])
    sts = list(scratch[NBUF:2 * NBUF])
    bt = scratch[2 * NBUF]
    gis = list(scratch[2 * NBUF + 1:2 * NBUF + 1 + NBUF])
    gos = list(scratch[2 * NBUF + 1 + NBUF:2 * NBUF + 1 + 2 * NBUF])
    wid = lax.axis_index("s") * NC + lax.axis_index("c")
    row_iota = lax.iota(jnp.int32, LANES)
    row_iota2 = row_iota + LANES

    def cb_of(k):
      return wid * per_w + k

    def copy_in(cb, b):
      return pltpu.make_async_copy(
          tt.at[:, pl.ds(cb * CB, CB)], bufs[b], gis[b])

    ones = jnp.full((LANES,), 1, jnp.int32)
    mask_cb = jnp.full((LANES,), CB - 1, jnp.int32)
    lanes16 = jnp.full((LANES,), LANES, jnp.int32)

    def transpose(b, st):
      buf = bufs[b]

      def vbody(j, col):
        for u in range(4):
          lo = plsc.load_gather(buf, [row_iota, col])
          hi = plsc.load_gather(buf, [row_iota2, col])
          idx = col * D + row_iota
          plsc.store_scatter(st, [idx], lo)
          plsc.store_scatter(st, [idx + lanes16], hi)
          col = (col + ones) & mask_cb
        return col

      lax.fori_loop(0, CB // 4, vbody, row_iota)

    def copy_out(cb, b):
      return pltpu.make_async_copy(
          sts[b], out.at[pl.ds(cb * CB * D, CB * D)], gos[b])

    n_mine = jnp.minimum(jnp.maximum(n_cb - wid * per_w, 0), per_w)

    @pl.when(n_mine > 0)
    def _():
      for b in range(NBUF):
        @pl.when(b < n_mine)
        def _(b=b):
          copy_in(cb_of(b), b).start()

      def step(k, b):
        @pl.when(k < n_mine)
        def _():
          copy_in(cb_of(k), b).wait()

          @pl.when(k >= NBUF)
          def _():
            copy_out(cb_of(k - NBUF), b).wait()
          transpose(b, sts[b])
          copy_out(cb_of(k), b).start()

          @pl.when(k + NBUF < n_mine)
          def _():
            copy_in(cb_of(k + NBUF), b).start()

      def body(j, _):
        for b in range(NBUF):
          step(j * NBUF + b, b)
        return 0

      lax.fori_loop(0, (per_w + NBUF - 1) // NBUF, body, 0)

      for back in range(1, NBUF + 1):
        k = n_mine - back

        @pl.when(k >= 0)
        def _(k=k):
          for b in range(NBUF):
            @pl.when(k % NBUF == b)
            def _(k=k, b=b):
              copy_out(cb_of(k), b).wait()

    if tail_w:
      @pl.when(wid == NW - 1)
      def _():
        pltpu.make_async_copy(tail, bt, gis[0]).start()
        pltpu.make_async_copy(tail, bt, gis[0]).wait()

        def body(r, _):
          sts[0][pl.ds(r * D, LANES)] = bt[r, pl.ds(0, LANES)]
          sts[0][pl.ds(r * D + LANES, LANES)] = bt[r, pl.ds(LANES, LANES)]
          return 0

        lax.fori_loop(0, tail_w, body, 0)
        pltpu.make_async_copy(
            sts[0].at[pl.ds(0, tail_w * D)],
            out.at[pl.ds(n_cb * CB * D, tail_w * D)], gos[0]).start()
        pltpu.make_async_copy(
            sts[0].at[pl.ds(0, tail_w * D)],
            out.at[pl.ds(n_cb * CB * D, tail_w * D)], gos[0]).wait()

  return relayout


def _epilogue_body(n_hist, B, pooled_ref, lbl_ref, w_ref, b_ref, out_ref):
  loss_sum = jnp.float32(0.0)
  correct = jnp.float32(0.0)
  ptp = jnp.float32(0.0); pfp = jnp.float32(0.0); pfn = jnp.float32(0.0)
  ntp = jnp.float32(0.0); nfp = jnp.float32(0.0); nfn = jnp.float32(0.0)
  eps = jnp.float32(EPS)
  for i in range(n_hist):
    pe = pooled_ref[i]
    sq = jnp.sum(pe * pe, axis=1, keepdims=True)
    normed = pe * lax.rsqrt(jnp.maximum(sq, 1e-12))
    logits = jnp.dot(normed, w_ref[i],
                     preferred_element_type=jnp.float32) + b_ref[i]
    p = jnp.clip(jax.nn.sigmoid(logits), eps, 1.0 - eps)
    lbl = lbl_ref[i]
    loss = -lbl * jnp.log(p) - (1.0 - lbl) * jnp.log(1.0 - p)
    loss_sum = loss_sum + jnp.sum(jnp.mean(loss, axis=0))

    pred_pos = p > 0.5
    is_pos = lbl == 1.0
    f32 = lambda x: jnp.asarray(x, jnp.float32)
    correct = correct + jnp.sum(f32(pred_pos == is_pos))
    ptp = ptp + jnp.sum(f32(jnp.logical_and(is_pos, pred_pos)))
    pfp = pfp + jnp.sum(f32(jnp.logical_and(~is_pos, pred_pos)))
    pfn = pfn + jnp.sum(f32(jnp.logical_and(is_pos, ~pred_pos)))

    pred_neg = p < 0.5
    is_neg = lbl == 0.0
    ntp = ntp + jnp.sum(f32(jnp.logical_and(is_neg, pred_neg)))
    nfp = nfp + jnp.sum(f32(jnp.logical_and(~is_neg, pred_neg)))
    nfn = nfn + jnp.sum(f32(jnp.logical_and(is_neg, ~pred_neg)))

  accuracy = correct / jnp.float32(B * 6 * n_hist)
  pos_recall = ptp / jnp.maximum(eps, ptp + pfn)
  pos_precision = ptp / jnp.maximum(eps, ptp + pfp)
  pos_f1 = 2 * pos_recall * pos_precision / jnp.maximum(
      eps, pos_recall + pos_precision)
  neg_recall = ntp / jnp.maximum(eps, ntp + nfn)
  neg_precision = ntp / jnp.maximum(eps, ntp + nfp)
  neg_f1 = 2 * neg_recall * neg_precision / jnp.maximum(
      eps, neg_recall + neg_precision)
  out_ref[0] = loss_sum
  out_ref[1] = (pos_f1 + neg_f1) / 2.0
  out_ref[2] = accuracy


def _epilogue_call(pooled, lbls, Ws, bs):
  n_hist, B, _ = pooled.shape
  return pl.pallas_call(
      functools.partial(_epilogue_body, n_hist, B),
      out_shape=jax.ShapeDtypeStruct((3,), jnp.float32),
      out_specs=pl.BlockSpec(memory_space=pltpu.SMEM),
  )(pooled, lbls, Ws, bs)


def kernel(unique_emb, history_0, history_1, history_2,
           label_0, label_1, label_2,
           W_0, W_1, W_2, b_0, b_1, b_2):
  V, D = unique_emb.shape
  B, Lh = history_0.shape
  hist = jnp.stack([history_0, history_1, history_2]).reshape(3 * B * Lh)
  n_full = (V // 128) * 128
  tail = lax.slice(unique_emb, (n_full, 0), (V, D))
  table = _make_relayout(V, D)(unique_emb.T, tail).reshape(V, D)
  pooled = _make_bag(V, D, B, Lh, 3)(table, hist).reshape(3, B, D)
  lbls = jnp.stack([label_0, label_1, label_2])
  Ws = jnp.stack([W_0, W_1, W_2])
  bs = jnp.stack([b_0, b_1, b_2])[:, None, :]
  o = _epilogue_call(pooled, lbls, Ws, bs)
  return (o[0], o[1], o[2])


# speedup vs baseline: 2.2059x; 1.0072x over previous
import functools

import jax
import jax.numpy as jnp
from jax import lax
from jax.experimental import pallas as pl
from jax.experimental.pallas import tpu as pltpu
from jax.experimental.pallas import tpu_sc as plsc

EPS = 1e-9


@functools.lru_cache(maxsize=None)
def _make_bag(V, D, B, Lh, n_hist):
  info = plsc.get_sparse_core_info()
  NC, NS, LANES = info.num_cores, info.num_subcores, info.num_lanes
  NW = NC * NS
  assert B % NW == 0 and D % LANES == 0
  b_per_w = B // NW
  n_bags = n_hist * b_per_w
  n_idx = n_bags * Lh
  C0 = min(128, Lh)
  C1 = Lh - C0
  assert Lh % 8 == 0

  mesh = plsc.VectorSubcoreMesh(core_axis_name="c", subcore_axis_name="s")
  NBUF = 4
  assert n_bags % NBUF == 0

  @functools.partial(
      pl.kernel,
      mesh=mesh,
      compiler_params=pltpu.CompilerParams(use_tc_tiling_on_sc=False),
      out_type=jax.ShapeDtypeStruct((n_hist * B, D), jnp.float32),
      scratch_types=(
          [pltpu.VMEM((n_idx,), jnp.int32)] +
          [pltpu.VMEM((Lh, D), jnp.float32) for _ in range(NBUF)] +
          [pltpu.VMEM((n_bags, D), jnp.float32)] +
          [pltpu.SemaphoreType.DMA for _ in range(NBUF)]
      ),
  )
  def bag(table, hist, out, *scr):
    idx_v = scr[0]
    bufs = list(scr[1:1 + NBUF])
    outv = scr[1 + NBUF]
    sems = list(scr[2 + NBUF:2 + 2 * NBUF])
    wid = lax.axis_index("s") * NC + lax.axis_index("c")
    for i in range(n_hist):
      pltpu.sync_copy(
          hist.at[pl.ds((i * B + wid * b_per_w) * Lh, b_per_w * Lh)],
          idx_v.at[pl.ds(i * b_per_w * Lh, b_per_w * Lh)],
      )

    def start(g, b):
      off = g * Lh
      buf = bufs[b]
      cps = [pltpu.make_async_copy(
          table.at[idx_v.at[pl.ds(off, C0)]], buf.at[pl.ds(0, C0)], sems[b])]
      if C1:
        cps.append(pltpu.make_async_copy(
            table.at[idx_v.at[pl.ds(off + C0, C1)]],
            buf.at[pl.ds(C0, C1)], sems[b]))
      return cps

    half = D // LANES

    def accum(g, b):
      buf = bufs[b]
      nacc = Lh // 8

      def body(j, carry):
        r = j * 8
        out_c = []
        for h in range(half):
          s = h * LANES
          v = (((buf[r, pl.ds(s, LANES)] + buf[r + 1, pl.ds(s, LANES)]) +
                (buf[r + 2, pl.ds(s, LANES)] + buf[r + 3, pl.ds(s, LANES)])) +
               ((buf[r + 4, pl.ds(s, LANES)] + buf[r + 5, pl.ds(s, LANES)]) +
                (buf[r + 6, pl.ds(s, LANES)] + buf[r + 7, pl.ds(s, LANES)])))
          out_c.append(carry[h] + v)
        return tuple(out_c)

      acc = lax.fori_loop(
          0, nacc, body,
          tuple(jnp.zeros((LANES,), jnp.float32) for _ in range(half)))
      for h in range(half):
        outv[g, pl.ds(h * LANES, LANES)] = acc[h]

    for b in range(NBUF):
      for c in start(b, b):
        c.start()

    def loop_body(j, _):
      for b in range(NBUF):
        k = j * NBUF + b
        for c in start(k, b):
          c.wait()
        accum(k, b)

        @pl.when(k + NBUF < n_bags)
        def _(k=k, b=b):
          for c in start(k + NBUF, b):
            c.start()

      return 0

    lax.fori_loop(0, n_bags // NBUF, loop_body, 0)

    for i in range(n_hist):
      pltpu.sync_copy(
          outv.at[pl.ds(i * b_per_w, b_per_w)],
          out.at[pl.ds(i * B + wid * b_per_w, b_per_w), :],
      )

  return bag


@functools.lru_cache(maxsize=None)
def _make_relayout(V, D):
  info = plsc.get_sparse_core_info()
  NC, NS, LANES = info.num_cores, info.num_subcores, info.num_lanes
  NW = NC * NS
  assert D == 2 * LANES
  CB = 128
  n_cb = V // CB
  per_w = -(-n_cb // NW)
  tail_w = V - n_cb * CB

  mesh = plsc.VectorSubcoreMesh(core_axis_name="c", subcore_axis_name="s")
  NBUF = 4

  @functools.partial(
      pl.kernel,
      mesh=mesh,
      compiler_params=pltpu.CompilerParams(needs_layout_passes=False),
      out_type=jax.ShapeDtypeStruct((V * D,), jnp.float32),
      scratch_types=(
          [pltpu.VMEM((D, CB), jnp.float32) for _ in range(NBUF)] +
          [pltpu.VMEM((CB * D,), jnp.float32) for _ in range(NBUF)] +
          [pltpu.VMEM((max(tail_w, 1), D), jnp.float32)] +
          [pltpu.SemaphoreType.DMA for _ in range(2 * NBUF)]
      ),
  )
  def relayout(tt, tail, out, *scratch):
    bufs = list(scratch[0:NBUF])
    sts = list(scratch[NBUF:2 * NBUF])
    bt = scratch[2 * NBUF]
    gis = list(scratch[2 * NBUF + 1:2 * NBUF + 1 + NBUF])
    gos = list(scratch[2 * NBUF + 1 + NBUF:2 * NBUF + 1 + 2 * NBUF])
    wid = lax.axis_index("s") * NC + lax.axis_index("c")
    row_iota = lax.iota(jnp.int32, LANES)
    row_iota2 = row_iota + LANES

    def cb_of(k):
      return wid * per_w + k

    def copy_in(cb, b):
      return pltpu.make_async_copy(
          tt.at[:, pl.ds(cb * CB, CB)], bufs[b], gis[b])

    ones = jnp.full((LANES,), 1, jnp.int32)
    mask_cb = jnp.full((LANES,), CB - 1, jnp.int32)
    lanes16 = jnp.full((LANES,), LANES, jnp.int32)

    def transpose(b, st):
      buf = bufs[b]

      def vbody(j, col):
        for u in range(8):
          lo = plsc.load_gather(buf, [row_iota, col])
          hi = plsc.load_gather(buf, [row_iota2, col])
          idx = col * D + row_iota
          plsc.store_scatter(st, [idx], lo)
          plsc.store_scatter(st, [idx + lanes16], hi)
          col = (col + ones) & mask_cb
        return col

      lax.fori_loop(0, CB // 8, vbody, row_iota)

    def copy_out(cb, b):
      return pltpu.make_async_copy(
          sts[b], out.at[pl.ds(cb * CB * D, CB * D)], gos[b])

    n_mine = jnp.minimum(jnp.maximum(n_cb - wid * per_w, 0), per_w)

    @pl.when(n_mine > 0)
    def _():
      for b in range(NBUF):
        @pl.when(b < n_mine)
        def _(b=b):
          copy_in(cb_of(b), b).start()

      def step(k, b):
        @pl.when(k < n_mine)
        def _():
          copy_in(cb_of(k), b).wait()

          @pl.when(k >= NBUF)
          def _():
            copy_out(cb_of(k - NBUF), b).wait()
          transpose(b, sts[b])
          copy_out(cb_of(k), b).start()

          @pl.when(k + NBUF < n_mine)
          def _():
            copy_in(cb_of(k + NBUF), b).start()

      def body(j, _):
        for b in range(NBUF):
          step(j * NBUF + b, b)
        return 0

      lax.fori_loop(0, (per_w + NBUF - 1) // NBUF, body, 0)

      for back in range(1, NBUF + 1):
        k = n_mine - back

        @pl.when(k >= 0)
        def _(k=k):
          for b in range(NBUF):
            @pl.when(k % NBUF == b)
            def _(k=k, b=b):
              copy_out(cb_of(k), b).wait()

    if tail_w:
      @pl.when(wid == NW - 1)
      def _():
        pltpu.make_async_copy(tail, bt, gis[0]).start()
        pltpu.make_async_copy(tail, bt, gis[0]).wait()

        def body(r, _):
          sts[0][pl.ds(r * D, LANES)] = bt[r, pl.ds(0, LANES)]
          sts[0][pl.ds(r * D + LANES, LANES)] = bt[r, pl.ds(LANES, LANES)]
          return 0

        lax.fori_loop(0, tail_w, body, 0)
        pltpu.make_async_copy(
            sts[0].at[pl.ds(0, tail_w * D)],
            out.at[pl.ds(n_cb * CB * D, tail_w * D)], gos[0]).start()
        pltpu.make_async_copy(
            sts[0].at[pl.ds(0, tail_w * D)],
            out.at[pl.ds(n_cb * CB * D, tail_w * D)], gos[0]).wait()

  return relayout


def _epilogue_body(n_hist, B, pooled_ref, lbl_ref, w_ref, b_ref, out_ref):
  loss_sum = jnp.float32(0.0)
  correct = jnp.float32(0.0)
  ptp = jnp.float32(0.0); pfp = jnp.float32(0.0); pfn = jnp.float32(0.0)
  ntp = jnp.float32(0.0); nfp = jnp.float32(0.0); nfn = jnp.float32(0.0)
  eps = jnp.float32(EPS)
  for i in range(n_hist):
    pe = pooled_ref[i]
    sq = jnp.sum(pe * pe, axis=1, keepdims=True)
    normed = pe * lax.rsqrt(jnp.maximum(sq, 1e-12))
    logits = jnp.dot(normed, w_ref[i],
                     preferred_element_type=jnp.float32) + b_ref[i]
    p = jnp.clip(jax.nn.sigmoid(logits), eps, 1.0 - eps)
    lbl = lbl_ref[i]
    loss = -lbl * jnp.log(p) - (1.0 - lbl) * jnp.log(1.0 - p)
    loss_sum = loss_sum + jnp.sum(jnp.mean(loss, axis=0))

    pred_pos = p > 0.5
    is_pos = lbl == 1.0
    f32 = lambda x: jnp.asarray(x, jnp.float32)
    correct = correct + jnp.sum(f32(pred_pos == is_pos))
    ptp = ptp + jnp.sum(f32(jnp.logical_and(is_pos, pred_pos)))
    pfp = pfp + jnp.sum(f32(jnp.logical_and(~is_pos, pred_pos)))
    pfn = pfn + jnp.sum(f32(jnp.logical_and(is_pos, ~pred_pos)))

    pred_neg = p < 0.5
    is_neg = lbl == 0.0
    ntp = ntp + jnp.sum(f32(jnp.logical_and(is_neg, pred_neg)))
    nfp = nfp + jnp.sum(f32(jnp.logical_and(~is_neg, pred_neg)))
    nfn = nfn + jnp.sum(f32(jnp.logical_and(is_neg, ~pred_neg)))

  accuracy = correct / jnp.float32(B * 6 * n_hist)
  pos_recall = ptp / jnp.maximum(eps, ptp + pfn)
  pos_precision = ptp / jnp.maximum(eps, ptp + pfp)
  pos_f1 = 2 * pos_recall * pos_precision / jnp.maximum(
      eps, pos_recall + pos_precision)
  neg_recall = ntp / jnp.maximum(eps, ntp + nfn)
  neg_precision = ntp / jnp.maximum(eps, ntp + nfp)
  neg_f1 = 2 * neg_recall * neg_precision / jnp.maximum(
      eps, neg_recall + neg_precision)
  out_ref[0] = loss_sum
  out_ref[1] = (pos_f1 + neg_f1) / 2.0
  out_ref[2] = accuracy


def _epilogue_call(pooled, lbls, Ws, bs):
  n_hist, B, _ = pooled.shape
  return pl.pallas_call(
      functools.partial(_epilogue_body, n_hist, B),
      out_shape=jax.ShapeDtypeStruct((3,), jnp.float32),
      out_specs=pl.BlockSpec(memory_space=pltpu.SMEM),
  )(pooled, lbls, Ws, bs)


def kernel(unique_emb, history_0, history_1, history_2,
           label_0, label_1, label_2,
           W_0, W_1, W_2, b_0, b_1, b_2):
  V, D = unique_emb.shape
  B, Lh = history_0.shape
  hist = jnp.stack([history_0, history_1, history_2]).reshape(3 * B * Lh)
  n_full = (V // 128) * 128
  tail = lax.slice(unique_emb, (n_full, 0), (V, D))
  table = _make_relayout(V, D)(unique_emb.T, tail).reshape(V, D)
  pooled = _make_bag(V, D, B, Lh, 3)(table, hist).reshape(3, B, D)
  lbls = jnp.stack([label_0, label_1, label_2])
  Ws = jnp.stack([W_0, W_1, W_2])
  bs = jnp.stack([b_0, b_1, b_2])[:, None, :]
  o = _epilogue_call(pooled, lbls, Ws, bs)
  return (o[0], o[1], o[2])
